# baseline (speedup 1.0000x reference)
import functools

import jax
import jax.numpy as jnp
from jax import lax
from jax.experimental import pallas as pl
from jax.experimental.pallas import tpu as pltpu
from jax.experimental.pallas import tpu_sc as plsc

LANES = 16
CHUNK = 128
NBUF = 4
DEPTH = 3
IBUF = 8


def _sc_geometry():
    try:
        info = plsc.get_sparse_core_info()
        return info.num_cores, info.num_subcores
    except Exception:
        return 2, 16


@functools.lru_cache(maxsize=None)
def _build(V, D, S, N, NC, NS):
    NW = NC * NS
    DP = 2 * D
    rows_per_w = N // NW
    n_chunks = rows_per_w // CHUNK
    assert n_chunks % IBUF == 0 and IBUF % NBUF == 0
    mesh = plsc.VectorSubcoreMesh(core_axis_name="c", subcore_axis_name="s")

    @functools.partial(
        pl.kernel,
        mesh=mesh,
        out_type=jax.ShapeDtypeStruct((N, D), jnp.float32),
        scratch_types=(
            [pltpu.VMEM((S, DP), jnp.float32)]
            + [pltpu.VMEM((CHUNK, DP), jnp.float32)] * NBUF
            + [pltpu.VMEM((CHUNK, D), jnp.float32)] * 2
            + [pltpu.VMEM((CHUNK,), jnp.int32)] * IBUF
            + [pltpu.SemaphoreType.DMA] * (2 * NBUF + IBUF)
        ),
    )
    def k(idx_hbm, table_hbm, pos_hbm, out_hbm, pos_v, *ring):
        rows = ring[:NBUF]
        narrow = ring[NBUF:NBUF + 2]
        idxb = ring[NBUF + 2:NBUF + 2 + IBUF]
        sems = ring[NBUF + 2 + IBUF:]
        gsem = sems[:NBUF]
        osem = sems[NBUF:2 * NBUF]
        isem = sems[2 * NBUF:]
        wid = lax.axis_index("s") * NC + lax.axis_index("c")
        base = wid * rows_per_w
        pltpu.sync_copy(pos_hbm, pos_v)

        def idx_copy(c, i):
            return pltpu.make_async_copy(
                idx_hbm.at[pl.ds(base + c * CHUNK, CHUNK)],
                idxb[i], isem[i])

        def gather(c, i, b):
            del c
            return pltpu.make_async_copy(
                table_hbm.at[idxb[i]], rows[b], gsem[b])

        def out_copy(c, b):
            return pltpu.make_async_copy(
                narrow[b % 2],
                out_hbm.at[pl.ds(base + c * CHUNK, CHUNK)], osem[b])

        for c in range(DEPTH):
            pltpu.sync_copy(idx_hbm.at[pl.ds(base + c * CHUNK, CHUNK)],
                            idxb[c % IBUF])
            gather(c, c % IBUF, c).start()
        for c in range(DEPTH, IBUF):
            if c < n_chunks:
                idx_copy(c, c % IBUF).start()

        def ring_body(q, carry):
            for u in range(IBUF):
                c = q * IBUF + u
                b = u % NBUF
                bg = (b + DEPTH) % NBUF
                ig = (u + DEPTH) % IBUF
                cg = c + DEPTH

                @pl.when(cg < n_chunks)
                def _():
                    @pl.when(cg >= NBUF)
                    def _():
                        out_copy(cg - NBUF, bg).wait()

                    idx_copy(cg, ig).wait()
                    gather(cg, ig, bg).start()

                gather(c, u, b).wait()

                @pl.when(c + IBUF < n_chunks)
                def _():
                    idx_copy(c + IBUF, u).start()

                @pl.when(c >= n_chunks - 2)
                def _(bp=(u + 2) % NBUF):
                    out_copy(c - 2, bp).wait()

                s_off = lax.rem(base + c * CHUNK, S)

                @plsc.parallel_loop(0, CHUNK, unroll=4)
                def add_body(j, b=b, s_off=s_off):
                    srow = s_off + j
                    srow = lax.select(srow >= S, srow - S, srow)
                    for kk in range(D // LANES):
                        pv = pos_v[srow, pl.ds(kk * LANES, LANES)]
                        tv = rows[b][j, pl.ds(kk * LANES, LANES)]
                        narrow[b % 2][j, pl.ds(kk * LANES, LANES)] = tv + pv
                out_copy(c, b).start()
            return carry

        lax.fori_loop(0, n_chunks // IBUF, ring_body, 0)
        for i in range(2):
            c_last = n_chunks - 2 + i
            out_copy(c_last, c_last % NBUF).wait()

    return k


_RBLK = 4096


@functools.lru_cache(maxsize=None)
def _transpose_pad(V, D):
    def body(in_ref, o_ref):
        xt = in_ref[...].T
        o_ref[:, pl.ds(0, D)] = xt
        o_ref[:, pl.ds(D, D)] = xt

    return pl.pallas_call(
        body,
        grid=((V + _RBLK - 1) // _RBLK,),
        in_specs=[pl.BlockSpec((D, _RBLK), lambda i: (0, i))],
        out_specs=pl.BlockSpec((_RBLK, 2 * D), lambda i: (i, 0)),
        out_shape=jax.ShapeDtypeStruct((V, 2 * D), jnp.float32),
    )


def kernel(inputs, token_table, pos_table):
    B, S = inputs.shape
    V, D = token_table.shape
    N = B * S
    NC, NS = _sc_geometry()
    idx = inputs.reshape(N).astype(jnp.int32)
    table128 = _transpose_pad(V, D)(token_table.T)
    pos128 = jnp.pad(pos_table, ((0, 0), (0, D)))
    out = _build(V, D, S, N, NC, NS)(idx, table128, pos128)
    return out.reshape(B, S, D)
